# Initial kernel scaffold; baseline (speedup 1.0000x reference)
import functools

import jax
import jax.numpy as jnp
from jax import lax
from jax.experimental import pallas as pl
from jax.experimental.pallas import tpu as pltpu
from jax.experimental.pallas import tpu_sc as plsc

_BATCH = 16384
_HIST = 50
_N = _BATCH * _HIST
_LANES = 128
_ROWS = _N // _LANES
_NW = 32
_ROWS_W = _ROWS // _NW


def _make_gather():
    mesh = plsc.VectorSubcoreMesh(core_axis_name="c", subcore_axis_name="s")

    @functools.partial(
        pl.kernel,
        mesh=mesh,
        out_type=jax.ShapeDtypeStruct((_ROWS, _LANES), jnp.float32),
        scratch_types=[
            pltpu.VMEM((_ROWS_W, _LANES), jnp.int32),
            pltpu.VMEM((_ROWS_W, _LANES), jnp.float32),
            pltpu.SemaphoreType.DMA,
        ],
    )
    def gather_kernel(x_hbm, tbl_hbm, out_hbm, idx_v, val_v, sem):
        wid = lax.axis_index("s") * 2 + lax.axis_index("c")
        base = wid * _ROWS_W
        pltpu.sync_copy(x_hbm.at[pl.ds(base, _ROWS_W)], idx_v)
        pltpu.async_copy(tbl_hbm.at[idx_v], val_v, sem).wait()
        pltpu.sync_copy(val_v, out_hbm.at[pl.ds(base, _ROWS_W)])

    return gather_kernel


def kernel(x, item_b):
    x32 = x.reshape(_ROWS, _LANES).astype(jnp.int32)
    out = _make_gather()(x32, item_b)
    return out.reshape(_BATCH, _HIST)


# speedup vs baseline: 1.3931x; 1.3931x over previous
import functools

import jax
import jax.numpy as jnp
from jax import lax
from jax.experimental import pallas as pl
from jax.experimental.pallas import tpu as pltpu
from jax.experimental.pallas import tpu_sc as plsc

_BATCH = 16384
_HIST = 50
_N = _BATCH * _HIST
_LANES = 128
_ROWS = _N // _LANES
_NW = 32
_ROWS_W = _ROWS // _NW


def _make_gather():
    mesh = plsc.VectorSubcoreMesh(core_axis_name="c", subcore_axis_name="s")

    @functools.partial(
        pl.kernel,
        mesh=mesh,
        out_type=jax.ShapeDtypeStruct((_ROWS, _LANES), jnp.float32),
        scratch_types=[
            pltpu.VMEM((_ROWS_W, _LANES), jnp.int32),
            pltpu.VMEM((_ROWS_W, _LANES), jnp.float32),
            pltpu.SemaphoreType.DMA,
        ],
    )
    def gather_kernel(x_hbm, tbl_hbm, out_hbm, idx_v, val_v, sem):
        wid = lax.axis_index("s") * 2 + lax.axis_index("c")
        base = wid * _ROWS_W
        pltpu.sync_copy(x_hbm.at[pl.ds(base, _ROWS_W)], idx_v)

        def fire(j, carry):
            pltpu.async_copy(tbl_hbm.at[idx_v.at[j]], val_v.at[j], sem)
            return carry

        lax.fori_loop(0, _ROWS_W, fire, 0)
        pltpu.make_async_copy(
            out_hbm.at[pl.ds(base, _ROWS_W)], val_v, sem
        ).wait()
        pltpu.sync_copy(val_v, out_hbm.at[pl.ds(base, _ROWS_W)])

    return gather_kernel


def kernel(x, item_b):
    x32 = x.reshape(_ROWS, _LANES).astype(jnp.int32)
    out = _make_gather()(x32, item_b)
    return out.reshape(_BATCH, _HIST)
